# baseline (speedup 1.0000x reference)
import jax
import jax.numpy as jnp
from jax import lax
from jax.experimental import pallas as pl
from jax.experimental.pallas import tpu as pltpu
from jax.experimental.pallas import tpu_sc as plsc

N = 50000
E = 1600000
NP = 50176
STRIPE = NP // 16
CB = 4
C = CB * 128
W = 32
NCHUNK = E // C
NCHUNK_A = 1563
NCHUNK_B = NCHUNK - NCHUNK_A


def _prep_body(vm_ref, va_ref, vre_ref, vim_ref):
    vm = vm_ref[...]
    va = va_ref[...]
    vre_ref[...] = vm * jnp.cos(va)
    vim_ref[...] = vm * jnp.sin(va)


def _make_sc_body(nchunk):
    base = nchunk // W
    rem = nchunk % W
    assert base % 2 == 0

    def _sc_body(ei2_h, ea2_h, p00, p01, p10, p11, vre_h, vim_h,
                 o00, o01, o10, o11,
                 vre, vim,
                 eib_0, attrb_0,
                 eib_1, attrb_1,
                 srcb2_0, dstb2_0, csr_0, csi_0, cdr_0, cdi_0,
                 srcb2_1, dstb2_1, csr_1, csi_1, cdr_1, cdi_1,
                 reb, imb, acc_re, acc_im,
                 vsem, isem0, isem1, ssem0, ssem1):
        cid = lax.axis_index("c")
        sid = lax.axis_index("s")
        w = cid * 16 + sid
        r0 = sid * STRIPE
        chunk0 = w * base

        INB = [(eib_0, attrb_0), (eib_1, attrb_1)]
        STG = [(srcb2_0, dstb2_0, csr_0, csi_0, cdr_0, cdi_0),
               (srcb2_1, dstb2_1, csr_1, csi_1, cdr_1, cdi_1)]
        ISEM = [isem0, isem1]
        SSEM = [ssem0, ssem1]

        def in_descs(g, p):
            eb, ab = INB[p]
            return [
                pltpu.make_async_copy(ei2_h.at[pl.ds(g * 8, 8)], eb, ISEM[p]),
                pltpu.make_async_copy(ea2_h.at[pl.ds(g * 8, 8)], ab, ISEM[p]),
            ]

        def sc_descs(p):
            s2, d2, cr, ci, dr_, di_ = STG[p]
            return [
                pltpu.make_async_copy(cr, acc_re.at[s2], SSEM[p]),
                pltpu.make_async_copy(ci, acc_im.at[s2], SSEM[p]),
                pltpu.make_async_copy(dr_, acc_re.at[d2], SSEM[p]),
                pltpu.make_async_copy(di_, acc_im.at[d2], SSEM[p]),
            ]

        def compute(p):
            eb, ab = INB[p]
            s2, d2, cr, ci, dr_, di_ = STG[p]

            @plsc.parallel_loop(0, C // 16)
            def j_body(t):
                blk = 2 * (t // 8)
                loff = (t % 8) * 16
                s = eb[blk, pl.ds(loff, 16)]
                d = eb[blk + 1, pl.ds(loff, 16)]
                yre = ab[blk, pl.ds(loff, 16)]
                yim = ab[blk + 1, pl.ds(loff, 16)]
                vsr = plsc.load_gather(vre, [s])
                vsi = plsc.load_gather(vim, [s])
                vdr = plsc.load_gather(vre, [d])
                vdi = plsc.load_gather(vim, [d])
                dre = vsr - vdr
                dim = vsi - vdi
                cre = yre * dre - yim * dim
                cim = yre * dim + yim * dre
                off = t * 16
                s2[pl.ds(off, 16)] = s
                d2[pl.ds(off, 16)] = d
                cr[pl.ds(off, 16)] = cre
                ci[pl.ds(off, 16)] = cim
                dr_[pl.ds(off, 16)] = -cre
                di_[pl.ds(off, 16)] = -cim

        vdesc = [pltpu.make_async_copy(vre_h.at[0], vre, vsem),
                 pltpu.make_async_copy(vim_h.at[0], vim, vsem)]
        for d in vdesc:
            d.start()
        for d in in_descs(chunk0, 0):
            d.start()
        for d in in_descs(chunk0 + 1, 1):
            d.start()

        @pl.when(cid == 0)
        def _():
            pltpu.sync_copy(p00.at[pl.ds(r0, STRIPE)], reb)
            pltpu.sync_copy(p01.at[pl.ds(r0, STRIPE)], imb)

        @pl.when(cid == 1)
        def _():
            pltpu.sync_copy(p10.at[pl.ds(r0, STRIPE)], reb)
            pltpu.sync_copy(p11.at[pl.ds(r0, STRIPE)], imb)

        pltpu.sync_copy(reb, acc_re.at[pl.ds(r0, STRIPE)])
        pltpu.sync_copy(imb, acc_im.at[pl.ds(r0, STRIPE)])
        for d in vdesc:
            d.wait()
        plsc.subcore_barrier()

        def phase(L, p):
            for d in in_descs(chunk0 + L, p):
                d.wait()

            @pl.when(L >= 2)
            def _():
                for d in sc_descs(p):
                    d.wait()

            compute(p)
            for d in sc_descs(p):
                d.start(add=True)

            @pl.when(L + 2 < base)
            def _():
                for d in in_descs(chunk0 + L + 2, p):
                    d.start()

        def body2(k2, carry):
            phase(2 * k2, 0)
            phase(2 * k2 + 1, 1)
            return carry

        lax.fori_loop(0, base // 2, body2, 0)
        for d in sc_descs(0):
            d.wait()
        for d in sc_descs(1):
            d.wait()

        @pl.when(w < rem)
        def _():
            g = W * base + w
            for d in in_descs(g, 0):
                d.start()
            for d in in_descs(g, 0):
                d.wait()
            compute(0)
            for d in sc_descs(0):
                d.start(add=True)
            for d in sc_descs(0):
                d.wait()

        plsc.subcore_barrier()

        pltpu.sync_copy(acc_re.at[pl.ds(r0, STRIPE)], reb)
        pltpu.sync_copy(acc_im.at[pl.ds(r0, STRIPE)], imb)

        @pl.when(cid == 0)
        def _():
            pltpu.sync_copy(reb, o00.at[pl.ds(r0, STRIPE)])
            pltpu.sync_copy(imb, o01.at[pl.ds(r0, STRIPE)])

        @pl.when(cid == 1)
        def _():
            pltpu.sync_copy(reb, o10.at[pl.ds(r0, STRIPE)])
            pltpu.sync_copy(imb, o11.at[pl.ds(r0, STRIPE)])

    return _sc_body


def _final_body(o00_ref, o01_ref, o10_ref, o11_ref, vre_ref, vim_ref,
                tre_ref, tim_ref, m_ref, out_ref):
    ire = o00_ref[...] + o10_ref[...]
    iim = o01_ref[...] + o11_ref[...]
    vre = vre_ref[...]
    vim = vim_ref[...]
    sre = vre * ire + vim * iim
    sim = vim * ire - vre * iim
    rre = sre - tre_ref[...]
    rim = sim - tim_ref[...]
    m = m_ref[...]
    rre = jnp.where(m, rre, 0.0)
    rim = jnp.where(m, rim, 0.0)
    a = jnp.sqrt(rre * rre + rim * rim)
    l0 = jnp.sum(a)
    l1 = jnp.sum(jnp.abs(rre))
    l2 = jnp.sum(jnp.abs(rim))
    lane = lax.broadcasted_iota(jnp.int32, (1, 128), 1)
    row = jnp.where(lane == 0, l0, jnp.where(lane == 1, l1,
                    jnp.where(lane == 2, l2, 0.0)))
    out_ref[...] = row * (1.0 / N)


def _make_sc_call(nchunk):
    return pl.kernel(
        _make_sc_body(nchunk),
        out_type=[jax.ShapeDtypeStruct((NP,), jnp.float32) for _ in range(4)],
        mesh=plsc.VectorSubcoreMesh(core_axis_name="c", subcore_axis_name="s",
                                    num_cores=2, num_subcores=16),
        compiler_params=pltpu.CompilerParams(needs_layout_passes=False),
        scratch_types=[
            pltpu.VMEM((NP,), jnp.float32),
            pltpu.VMEM((NP,), jnp.float32),
            pltpu.VMEM((8, 128), jnp.int32),
            pltpu.VMEM((8, 128), jnp.float32),
            pltpu.VMEM((8, 128), jnp.int32),
            pltpu.VMEM((8, 128), jnp.float32),
            pltpu.VMEM((C,), jnp.int32),
            pltpu.VMEM((C,), jnp.int32),
            pltpu.VMEM((C,), jnp.float32),
            pltpu.VMEM((C,), jnp.float32),
            pltpu.VMEM((C,), jnp.float32),
            pltpu.VMEM((C,), jnp.float32),
            pltpu.VMEM((C,), jnp.int32),
            pltpu.VMEM((C,), jnp.int32),
            pltpu.VMEM((C,), jnp.float32),
            pltpu.VMEM((C,), jnp.float32),
            pltpu.VMEM((C,), jnp.float32),
            pltpu.VMEM((C,), jnp.float32),
            pltpu.VMEM((STRIPE,), jnp.float32),
            pltpu.VMEM((STRIPE,), jnp.float32),
            pltpu.VMEM_SHARED((NP,), jnp.float32),
            pltpu.VMEM_SHARED((NP,), jnp.float32),
            pltpu.SemaphoreType.DMA,
            pltpu.SemaphoreType.DMA,
            pltpu.SemaphoreType.DMA,
            pltpu.SemaphoreType.DMA,
            pltpu.SemaphoreType.DMA,
        ],
    )


_sc_call_a = _make_sc_call(NCHUNK_A)
_sc_call_b = _make_sc_call(NCHUNK_B)


def kernel(pred, target, edge_index, edge_attr, mask):
    pad = (0, NP - N)
    vm = jnp.pad(pred[:, 0], pad).reshape(1, NP)
    va = jnp.pad(pred[:, 1], pad).reshape(1, NP)
    tre = jnp.pad(target[:, 0], pad).reshape(1, NP)
    tim = jnp.pad(target[:, 1], pad).reshape(1, NP)
    mp = jnp.pad(mask, pad).reshape(1, NP)
    BA = NCHUNK_A * 4
    ei3 = edge_index.reshape(2, E // 128, 128)
    ea3 = edge_attr.reshape(E // 128, 128, 2)
    ei2a = ei3[:, :BA].transpose(1, 0, 2).reshape(8 * NCHUNK_A, 128)
    ei2b = ei3[:, BA:].transpose(1, 0, 2).reshape(8 * NCHUNK_B, 128)
    ea2a = ea3[:BA].transpose(0, 2, 1).reshape(8 * NCHUNK_A, 128)
    ea2b = ea3[BA:].transpose(0, 2, 1).reshape(8 * NCHUNK_B, 128)
    pz = jnp.zeros((NP,), jnp.float32)

    vre_h, vim_h = pl.pallas_call(
        _prep_body,
        out_shape=[jax.ShapeDtypeStruct((1, NP), jnp.float32)] * 2,
    )(vm, va)

    a00, a01, a10, a11 = _sc_call_a(ei2a, ea2a, pz, pz, pz, pz, vre_h, vim_h)
    o00, o01, o10, o11 = _sc_call_b(ei2b, ea2b, a00, a01, a10, a11,
                                    vre_h, vim_h)

    out = pl.pallas_call(
        _final_body,
        out_shape=jax.ShapeDtypeStruct((1, 128), jnp.float32),
    )(o00.reshape(1, NP), o01.reshape(1, NP), o10.reshape(1, NP),
      o11.reshape(1, NP), vre_h, vim_h, tre, tim, mp)
    return out[0, :3]


# speedup vs baseline: 1.3204x; 1.3204x over previous
import jax
import jax.numpy as jnp
from jax import lax
from jax.experimental import pallas as pl
from jax.experimental.pallas import tpu as pltpu
from jax.experimental.pallas import tpu_sc as plsc

N = 50000
E = 1600000
NP = 50176
STRIPE = NP // 16
CB = 4
C = CB * 128
W = 32
NCHUNK = E // C
CHUNK_BASE = NCHUNK // W
CHUNK_REM = NCHUNK % W


def _prep_body(vm_ref, va_ref, vre_ref, vim_ref):
    vm = vm_ref[...]
    va = va_ref[...]
    vre_ref[...] = vm * jnp.cos(va)
    vim_ref[...] = vm * jnp.sin(va)


def _sc_body(ei2_h, ea2_h, vre_h, vim_h, zsm,
             o00, o01, o10, o11,
             vre, vim,
             eib_0, attrb_0,
             eib_1, attrb_1,
             srcb2_0, dstb2_0, csr_0, csi_0, cdr_0, cdi_0,
             srcb2_1, dstb2_1, csr_1, csi_1, cdr_1, cdi_1,
             reb, imb, acc_re, acc_im,
             vsem, isem0, isem1, ssem0, ssem1):
    cid = lax.axis_index("c")
    sid = lax.axis_index("s")
    w = cid * 16 + sid
    r0 = sid * STRIPE
    iota = lax.iota(jnp.int32, 16)
    chunk0 = w * CHUNK_BASE

    INB = [(eib_0, attrb_0), (eib_1, attrb_1)]
    STG = [(srcb2_0, dstb2_0, csr_0, csi_0, cdr_0, cdi_0),
           (srcb2_1, dstb2_1, csr_1, csi_1, cdr_1, cdi_1)]
    ISEM = [isem0, isem1]
    SSEM = [ssem0, ssem1]

    def in_descs(g, p):
        eb, ab = INB[p]
        return [
            pltpu.make_async_copy(ei2_h.at[pl.ds(g * 8, 8)], eb, ISEM[p]),
            pltpu.make_async_copy(ea2_h.at[pl.ds(g * 8, 8)], ab, ISEM[p]),
        ]

    def sc_descs(p):
        s2, d2, cr, ci, dr_, di_ = STG[p]
        return [
            pltpu.make_async_copy(cr, acc_re.at[s2], SSEM[p]),
            pltpu.make_async_copy(ci, acc_im.at[s2], SSEM[p]),
            pltpu.make_async_copy(dr_, acc_re.at[d2], SSEM[p]),
            pltpu.make_async_copy(di_, acc_im.at[d2], SSEM[p]),
        ]

    def compute(p):
        eb, ab = INB[p]
        s2, d2, cr, ci, dr_, di_ = STG[p]

        @plsc.parallel_loop(0, C // 16)
        def j_body(t):
            blk = 2 * (t // 8)
            loff = (t % 8) * 16
            s = eb[blk, pl.ds(loff, 16)]
            d = eb[blk + 1, pl.ds(loff, 16)]
            yre = ab[blk, pl.ds(loff, 16)]
            yim = ab[blk + 1, pl.ds(loff, 16)]
            vsr = plsc.load_gather(vre, [s])
            vsi = plsc.load_gather(vim, [s])
            vdr = plsc.load_gather(vre, [d])
            vdi = plsc.load_gather(vim, [d])
            dre = vsr - vdr
            dim = vsi - vdi
            cre = yre * dre - yim * dim
            cim = yre * dim + yim * dre
            off = t * 16
            s2[pl.ds(off, 16)] = s
            d2[pl.ds(off, 16)] = d
            cr[pl.ds(off, 16)] = cre
            ci[pl.ds(off, 16)] = cim
            dr_[pl.ds(off, 16)] = -cre
            di_[pl.ds(off, 16)] = -cim

    vdesc = [pltpu.make_async_copy(vre_h.at[0], vre, vsem),
             pltpu.make_async_copy(vim_h.at[0], vim, vsem)]
    for d in vdesc:
        d.start()
    for d in in_descs(chunk0, 0):
        d.start()
    for d in in_descs(chunk0 + 1, 1):
        d.start()
    pltpu.sync_copy(zsm, reb)
    pltpu.sync_copy(reb, acc_re.at[pl.ds(r0, STRIPE)])
    pltpu.sync_copy(reb, acc_im.at[pl.ds(r0, STRIPE)])
    for d in vdesc:
        d.wait()
    plsc.subcore_barrier()

    def phase(L, p):
        for d in in_descs(chunk0 + L, p):
            d.wait()

        @pl.when(L >= 2)
        def _():
            for d in sc_descs(p):
                d.wait()

        compute(p)
        for d in sc_descs(p):
            d.start(add=True)

        @pl.when(L + 2 < CHUNK_BASE)
        def _():
            for d in in_descs(chunk0 + L + 2, p):
                d.start()

    def body2(k2, carry):
        phase(2 * k2, 0)
        phase(2 * k2 + 1, 1)
        return carry

    lax.fori_loop(0, CHUNK_BASE // 2, body2, 0)
    if CHUNK_BASE % 2:
        phase(jnp.int32(CHUNK_BASE - 1), 0)
    for d in sc_descs(0):
        d.wait()
    for d in sc_descs(1):
        d.wait()

    @pl.when(w < CHUNK_REM)
    def _():
        g = W * CHUNK_BASE + w
        for d in in_descs(g, 0):
            d.start()
        for d in in_descs(g, 0):
            d.wait()
        compute(0)
        for d in sc_descs(0):
            d.start(add=True)
        for d in sc_descs(0):
            d.wait()

    plsc.subcore_barrier()

    pltpu.sync_copy(acc_re.at[pl.ds(r0, STRIPE)], reb)
    pltpu.sync_copy(acc_im.at[pl.ds(r0, STRIPE)], imb)

    @pl.when(cid == 0)
    def _():
        pltpu.sync_copy(reb, o00.at[pl.ds(r0, STRIPE)])
        pltpu.sync_copy(imb, o01.at[pl.ds(r0, STRIPE)])

    @pl.when(cid == 1)
    def _():
        pltpu.sync_copy(reb, o10.at[pl.ds(r0, STRIPE)])
        pltpu.sync_copy(imb, o11.at[pl.ds(r0, STRIPE)])


def _final_body(o00_ref, o01_ref, o10_ref, o11_ref, vre_ref, vim_ref,
                tre_ref, tim_ref, m_ref, out_ref):
    ire = o00_ref[...] + o10_ref[...]
    iim = o01_ref[...] + o11_ref[...]
    vre = vre_ref[...]
    vim = vim_ref[...]
    sre = vre * ire + vim * iim
    sim = vim * ire - vre * iim
    rre = sre - tre_ref[...]
    rim = sim - tim_ref[...]
    m = m_ref[...]
    rre = jnp.where(m, rre, 0.0)
    rim = jnp.where(m, rim, 0.0)
    a = jnp.sqrt(rre * rre + rim * rim)
    l0 = jnp.sum(a)
    l1 = jnp.sum(jnp.abs(rre))
    l2 = jnp.sum(jnp.abs(rim))
    lane = lax.broadcasted_iota(jnp.int32, (1, 128), 1)
    row = jnp.where(lane == 0, l0, jnp.where(lane == 1, l1,
                    jnp.where(lane == 2, l2, 0.0)))
    out_ref[...] = row * (1.0 / N)


_sc_call = pl.kernel(
    _sc_body,
    out_type=[jax.ShapeDtypeStruct((NP,), jnp.float32) for _ in range(4)],
    mesh=plsc.VectorSubcoreMesh(core_axis_name="c", subcore_axis_name="s",
                                num_cores=2, num_subcores=16),
    compiler_params=pltpu.CompilerParams(needs_layout_passes=False),
    scratch_types=[
        pltpu.VMEM((NP,), jnp.float32),
        pltpu.VMEM((NP,), jnp.float32),
        pltpu.VMEM((8, 128), jnp.int32),
        pltpu.VMEM((8, 128), jnp.float32),
        pltpu.VMEM((8, 128), jnp.int32),
        pltpu.VMEM((8, 128), jnp.float32),
        pltpu.VMEM((C,), jnp.int32),
        pltpu.VMEM((C,), jnp.int32),
        pltpu.VMEM((C,), jnp.float32),
        pltpu.VMEM((C,), jnp.float32),
        pltpu.VMEM((C,), jnp.float32),
        pltpu.VMEM((C,), jnp.float32),
        pltpu.VMEM((C,), jnp.int32),
        pltpu.VMEM((C,), jnp.int32),
        pltpu.VMEM((C,), jnp.float32),
        pltpu.VMEM((C,), jnp.float32),
        pltpu.VMEM((C,), jnp.float32),
        pltpu.VMEM((C,), jnp.float32),
        pltpu.VMEM((STRIPE,), jnp.float32),
        pltpu.VMEM((STRIPE,), jnp.float32),
        pltpu.VMEM_SHARED((NP,), jnp.float32),
        pltpu.VMEM_SHARED((NP,), jnp.float32),
        pltpu.SemaphoreType.DMA,
        pltpu.SemaphoreType.DMA,
        pltpu.SemaphoreType.DMA,
        pltpu.SemaphoreType.DMA,
        pltpu.SemaphoreType.DMA,
    ],
)


def kernel(pred, target, edge_index, edge_attr, mask):
    pad = (0, NP - N)
    vm = jnp.pad(pred[:, 0], pad).reshape(1, NP)
    va = jnp.pad(pred[:, 1], pad).reshape(1, NP)
    tre = jnp.pad(target[:, 0], pad).reshape(1, NP)
    tim = jnp.pad(target[:, 1], pad).reshape(1, NP)
    mp = jnp.pad(mask, pad).reshape(1, NP)
    ei2 = (edge_index.reshape(2, E // 128, 128)
           .transpose(1, 0, 2).reshape(E // 64, 128))
    ea2 = (edge_attr.reshape(E // 128, 128, 2)
           .transpose(0, 2, 1).reshape(E // 64, 128))
    zsm = jnp.zeros((STRIPE,), jnp.float32)

    vre_h, vim_h = pl.pallas_call(
        _prep_body,
        out_shape=[jax.ShapeDtypeStruct((1, NP), jnp.float32)] * 2,
    )(vm, va)

    o00, o01, o10, o11 = _sc_call(ei2, ea2, vre_h, vim_h, zsm)

    out = pl.pallas_call(
        _final_body,
        out_shape=jax.ShapeDtypeStruct((1, 128), jnp.float32),
    )(o00.reshape(1, NP), o01.reshape(1, NP), o10.reshape(1, NP),
      o11.reshape(1, NP), vre_h, vim_h, tre, tim, mp)
    return out[0, :3]
